# baseline (speedup 1.0000x reference)
import jax
import jax.numpy as jnp
from jax.experimental import pallas as pl
from jax.experimental.pallas import tpu as pltpu

_N = 10000
_R = 200
_NBLK = _N // _R
_NPASS = 4
_BETA = 0.7

_F32 = jnp.float32
_HI = jax.lax.Precision.HIGHEST


def _dot_nn(a, b, prec=None):
    return jax.lax.dot_general(a, b, (((1,), (0,)), ((), ())),
                               preferred_element_type=_F32, precision=prec)


def _dot_nt(a, b, prec=None):
    return jax.lax.dot_general(a, b, (((1,), (1,)), ((), ())),
                               preferred_element_type=_F32, precision=prec)


def _hitsgnn_kernel(adj_ref, x_ref, w1_ref, b1_ref, w2_ref, b2_ref,
                    y2_out, auth_out, hub_out,
                    s1_s, y1_s, s2_s, y2_s,
                    r_s, c_s, t1_s, u1_s, t2_s, u2_s, auth_s):
    p = pl.program_id(0)
    i = pl.program_id(1)
    rows = pl.ds(i * _R, _R)
    ones_n = jnp.ones((1, _N), dtype=_F32)
    ones_r = jnp.ones((1, _R), dtype=_F32)

    @pl.when(jnp.logical_and(p == 0, i == 0))
    def _init0():
        s1_s[...] = _dot_nn(x_ref[...], w1_ref[...])
        c_s[...] = jnp.zeros_like(c_s)

    @pl.when(p == 0)
    def _pass0():
        adj = adj_ref[...]
        y1_s[rows, :] = _dot_nn(adj, s1_s[...])
        r_s[i, 0:1, :] = _dot_nt(ones_n, adj, _HI)
        c_s[...] += _dot_nn(ones_r, adj, _HI)

    @pl.when(jnp.logical_and(p == 1, i == 0))
    def _init1():
        h = jnp.maximum(y1_s[...] + b1_ref[...], 0.0)
        s2_s[...] = _dot_nn(h, w2_ref[...])
        t1_s[...] = jnp.zeros_like(t1_s)

    @pl.when(p == 1)
    def _pass1():
        adj = adj_ref[...]
        y2_s[rows, :] = _dot_nn(adj, s2_s[...])
        u1_s[i, 0:1, :] = _dot_nt(c_s[...], adj, _HI)
        t1_s[...] += _dot_nn(r_s[i, 0:1, :], adj, _HI)

    @pl.when(jnp.logical_and(p == 2, i == 0))
    def _init2():
        u2_s[...] = jnp.zeros_like(u2_s)

    @pl.when(p == 2)
    def _pass2():
        adj = adj_ref[...]
        t2_s[i, 0:1, :] = _dot_nt(t1_s[...], adj, _HI)
        u2_s[...] += _dot_nn(u1_s[i, 0:1, :], adj, _HI)

    @pl.when(jnp.logical_and(p == 3, i == 0))
    def _init3():
        auth_s[...] = jnp.zeros_like(auth_s)

    @pl.when(p == 3)
    def _pass3():
        adj = adj_ref[...]
        hub_out[0, 0:1, :] = _dot_nt(u2_s[...], adj, _HI)
        auth_s[...] += _dot_nn(t2_s[i, 0:1, :], adj, _HI)
        y2_out[...] = y2_s[rows, :]

    @pl.when(jnp.logical_and(p == _NPASS - 1, i == _NBLK - 1))
    def _emit_auth():
        auth_out[...] = auth_s[...]


def _finale_kernel(auth_ref, hub_ref, y2_ref, b2_ref, out_ref):
    score = auth_ref[...] + hub_ref[...]
    e = jnp.exp(score - jnp.max(score))
    hits = e / jnp.sum(e)
    out = y2_ref[...] + b2_ref[...]
    xx = out
    for _ in range(2):
        s = _dot_nn(hits, xx, _HI)
        xx = (1.0 - _BETA) * s + _BETA * out
    mx = jnp.max(xx, axis=1, keepdims=True)
    lse = jnp.log(jnp.sum(jnp.exp(xx - mx), axis=1, keepdims=True)) + mx
    out_ref[...] = xx - lse


def kernel(x, adj, W1, b1, W2, b2):
    nfeat = x.shape[1]
    nhid = W1.shape[1]
    nclass = W2.shape[1]
    b1r = b1.reshape(1, nhid)
    b2r = b2.reshape(1, nclass)
    blkvec = pltpu.VMEM((_NBLK, 1, _R), _F32)
    fullvec = pltpu.VMEM((1, _N), _F32)
    y2, auth, hub = pl.pallas_call(
        _hitsgnn_kernel,
        grid=(_NPASS, _NBLK),
        in_specs=[
            pl.BlockSpec((_R, _N), lambda p, i: (i, 0)),
            pl.BlockSpec((_N, nfeat), lambda p, i: (0, 0)),
            pl.BlockSpec((nfeat, nhid), lambda p, i: (0, 0)),
            pl.BlockSpec((1, nhid), lambda p, i: (0, 0)),
            pl.BlockSpec((nhid, nclass), lambda p, i: (0, 0)),
            pl.BlockSpec((1, nclass), lambda p, i: (0, 0)),
        ],
        out_specs=[
            pl.BlockSpec((_R, nclass), lambda p, i: (i, 0)),
            pl.BlockSpec((1, _N), lambda p, i: (0, 0)),
            pl.BlockSpec((1, 1, _R), lambda p, i: (i, 0, 0)),
        ],
        out_shape=[
            jax.ShapeDtypeStruct((_N, nclass), _F32),
            jax.ShapeDtypeStruct((1, _N), _F32),
            jax.ShapeDtypeStruct((_NBLK, 1, _R), _F32),
        ],
        scratch_shapes=[
            pltpu.VMEM((_N, nhid), _F32),
            pltpu.VMEM((_N, nhid), _F32),
            pltpu.VMEM((_N, nclass), _F32),
            pltpu.VMEM((_N, nclass), _F32),
            blkvec,
            fullvec,
            fullvec,
            blkvec,
            blkvec,
            fullvec,
            fullvec,
        ],
    )(adj, x, W1, b1r, W2, b2r)
    hub_full = hub.reshape(1, _N)
    return pl.pallas_call(
        _finale_kernel,
        in_specs=[
            pl.BlockSpec((1, _N), lambda: (0, 0)),
            pl.BlockSpec((1, _N), lambda: (0, 0)),
            pl.BlockSpec((_N, nclass), lambda: (0, 0)),
            pl.BlockSpec((1, nclass), lambda: (0, 0)),
        ],
        out_specs=pl.BlockSpec((_N, nclass), lambda: (0, 0)),
        out_shape=jax.ShapeDtypeStruct((_N, nclass), _F32),
    )(auth, hub_full, y2, b2r)


# speedup vs baseline: 4.1480x; 4.1480x over previous
import jax
import jax.numpy as jnp
from jax.experimental import pallas as pl
from jax.experimental.pallas import tpu as pltpu

_N = 10000
_R = 200
_NBLK = _N // _R
_NPASS = 3
_BETA = 0.7

_F32 = jnp.float32
_BF16 = jnp.bfloat16
_HI = jax.lax.Precision.HIGHEST


def _dot_nn(a, b, prec=None):
    return jax.lax.dot_general(a, b, (((1,), (0,)), ((), ())),
                               preferred_element_type=_F32, precision=prec)


def _hitsgnn_kernel(adj_ref, x_ref, w1_ref, b1_ref, w2_ref, b2_ref,
                    y2_out, auth_out, hub_out,
                    s1_s, y1_s, s2_s, y2_s,
                    c_s, t1_s, u2_s, auth_s):
    p = pl.program_id(0)
    i = pl.program_id(1)
    rows = pl.ds(i * _R, _R)

    @pl.when(jnp.logical_and(p == 0, i == 0))
    def _init0():
        s1_s[...] = _dot_nn(x_ref[...], w1_ref[...])
        c_s[...] = jnp.zeros_like(c_s)
        t1_s[...] = jnp.zeros_like(t1_s)

    @pl.when(p == 0)
    def _pass0():
        a = adj_ref[...]
        y1_s[rows, :] = _dot_nn(a.astype(_BF16), s1_s[...].astype(_BF16))
        c_s[...] += jnp.sum(a, axis=0, keepdims=True)
        r_col = jnp.sum(a, axis=1, keepdims=True)
        t1_s[...] += jnp.sum(a * r_col, axis=0, keepdims=True)

    @pl.when(jnp.logical_and(p == 1, i == 0))
    def _init1():
        h = jnp.maximum(y1_s[...] + b1_ref[...], 0.0)
        s2_s[...] = _dot_nn(h, w2_ref[...])
        u2_s[...] = jnp.zeros_like(u2_s)
        auth_s[...] = jnp.zeros_like(auth_s)

    @pl.when(p == 1)
    def _pass1():
        a = adj_ref[...]
        y2_s[rows, :] = _dot_nn(a.astype(_BF16), s2_s[...].astype(_BF16))
        t2_col = jnp.sum(a * t1_s[...], axis=1, keepdims=True)
        auth_s[...] += jnp.sum(a * t2_col, axis=0, keepdims=True)
        u1_col = jnp.sum(a * c_s[...], axis=1, keepdims=True)
        u2_s[...] += jnp.sum(a * u1_col, axis=0, keepdims=True)

    @pl.when(p == 2)
    def _pass2():
        a = adj_ref[...]
        hub_out[...] = jnp.sum(a * u2_s[...], axis=1, keepdims=True)
        y2_out[...] = y2_s[rows, :]

    @pl.when(jnp.logical_and(p == _NPASS - 1, i == _NBLK - 1))
    def _emit_auth():
        auth_out[...] = auth_s[...]


def _finale_kernel(auth_ref, hub_ref, y2_ref, b2_ref, out_ref):
    score = auth_ref[...] + hub_ref[...]
    e = jnp.exp(score - jnp.max(score))
    hits = e / jnp.sum(e)
    out = y2_ref[...] + b2_ref[...]
    xx = out
    for _ in range(2):
        s = _dot_nn(hits, xx, _HI)
        xx = (1.0 - _BETA) * s + _BETA * out
    mx = jnp.max(xx, axis=1, keepdims=True)
    lse = jnp.log(jnp.sum(jnp.exp(xx - mx), axis=1, keepdims=True)) + mx
    out_ref[...] = xx - lse


def kernel(x, adj, W1, b1, W2, b2):
    nfeat = x.shape[1]
    nhid = W1.shape[1]
    nclass = W2.shape[1]
    b1r = b1.reshape(1, nhid)
    b2r = b2.reshape(1, nclass)
    fullvec = pltpu.VMEM((1, _N), _F32)
    y2, auth, hub = pl.pallas_call(
        _hitsgnn_kernel,
        grid=(_NPASS, _NBLK),
        in_specs=[
            pl.BlockSpec((_R, _N), lambda p, i: (i, 0)),
            pl.BlockSpec((_N, nfeat), lambda p, i: (0, 0)),
            pl.BlockSpec((nfeat, nhid), lambda p, i: (0, 0)),
            pl.BlockSpec((1, nhid), lambda p, i: (0, 0)),
            pl.BlockSpec((nhid, nclass), lambda p, i: (0, 0)),
            pl.BlockSpec((1, nclass), lambda p, i: (0, 0)),
        ],
        out_specs=[
            pl.BlockSpec((_R, nclass), lambda p, i: (i, 0)),
            pl.BlockSpec((1, _N), lambda p, i: (0, 0)),
            pl.BlockSpec((_R, 1), lambda p, i: (i, 0)),
        ],
        out_shape=[
            jax.ShapeDtypeStruct((_N, nclass), _F32),
            jax.ShapeDtypeStruct((1, _N), _F32),
            jax.ShapeDtypeStruct((_N, 1), _F32),
        ],
        scratch_shapes=[
            pltpu.VMEM((_N, nhid), _F32),
            pltpu.VMEM((_N, nhid), _F32),
            pltpu.VMEM((_N, nclass), _F32),
            pltpu.VMEM((_N, nclass), _F32),
            fullvec,
            fullvec,
            fullvec,
            fullvec,
        ],
    )(adj, x, W1, b1r, W2, b2r)
    hub_full = hub.reshape(1, _N)
    return pl.pallas_call(
        _finale_kernel,
        in_specs=[
            pl.BlockSpec((1, _N), lambda: (0, 0)),
            pl.BlockSpec((1, _N), lambda: (0, 0)),
            pl.BlockSpec((_N, nclass), lambda: (0, 0)),
            pl.BlockSpec((1, nclass), lambda: (0, 0)),
        ],
        out_specs=pl.BlockSpec((_N, nclass), lambda: (0, 0)),
        out_shape=jax.ShapeDtypeStruct((_N, nclass), _F32),
    )(auth, hub_full, y2, b2r)
